# baseline (speedup 1.0000x reference)
import functools

import jax
import jax.numpy as jnp
from jax import lax
from jax.experimental import pallas as pl
from jax.experimental.pallas import tpu as pltpu
from jax.experimental.pallas import tpu_sc as plsc

_POS_TH = 2.0
_B = 8
_N = 512 * 512
_M = _N // 4
_CHUNK = 8192
_NCH = _M // _CHUNK
_CV = _CHUNK // 16
_NV = _M // 16
_NB = 256
_SHIFTS = (23, 15, 7, 0)
_LBITS = (8, 8, 8, 7)


def _sc_body(x_hbm, y_hbm, out_hbm, nb, xb, yb, hist, shist, comb,
             thist, tshist, tcf, statv, outv, sh_all):
    cid = lax.axis_index("c")
    sid = lax.axis_index("s")
    sample = cid * 4 + sid // 4
    quarter = sid % 4
    q0 = (sid // 4) * 4
    base = pl.multiple_of(sample * _N + quarter * _M, 8)

    iot = lax.iota(jnp.int32, 16)
    zero_i = jnp.zeros((16,), jnp.int32)
    zero_f = jnp.zeros((16,), jnp.float32)
    ones_i = jnp.ones((16,), jnp.int32)

    def zero_body(j):
        for u in range(4):
            hist[pl.ds(j + u * 16, 16)] = zero_i
            shist[pl.ds(j + u * 16, 16)] = zero_f

    plsc.parallel_loop(0, _NB * 16, step=64)(zero_body)

    _U = 4
    sh0 = jnp.full((16,), _SHIFTS[0], jnp.int32)
    apc = (zero_i,) * _U
    for c in range(_NCH):
        off = pl.multiple_of(base + c * _CHUNK, 8)
        pltpu.sync_copy(x_hbm.at[pl.ds(off, _CHUNK)], xb)
        pltpu.sync_copy(y_hbm.at[pl.ds(off, _CHUNK)], yb)

        def p1_body(i, acc, c=c):
            out = []
            for u in range(_U):
                s = i + u * 16
                xv = xb[pl.ds(s, 16)]
                yv = yb[pl.ds(s, 16)]
                d = xv - yv
                lv = d * d
                posm = yv > _POS_TH
                nbv = jnp.where(
                    posm, zero_i, lax.bitcast_convert_type(lv, jnp.int32)
                )
                nb[pl.ds(c * _CHUNK + s, 16)] = nbv
                idx = iot * _NB + lax.shift_right_logical(nbv, sh0)
                plsc.addupdate_scatter(hist, [idx], ones_i)
                plsc.addupdate_scatter(shist, [idx], lv)
                out.append(acc[u] + jnp.where(posm, ones_i, zero_i))
            return tuple(out)

        apc = plsc.parallel_loop(0, _CHUNK, step=16 * _U, carry=apc)(p1_body)
    ap = apc[0] + apc[1] + apc[2] + apc[3]

    def lane_reduce(nbins):
        def body(j, _):
            acc_c = zero_i
            acc_s = zero_f
            for l in range(16):
                acc_c = acc_c + hist[pl.ds(l * _NB + j * 16, 16)]
                acc_s = acc_s + shist[pl.ds(l * _NB + j * 16, 16)]
            thist[pl.ds(j * 16, 16)] = acc_c
            tshist[pl.ds(j * 16, 16)] = acc_s
            return 0

        lax.fori_loop(0, nbins // 16, body, 0)

    def publish_combine(nbins):
        def cvt(j, _):
            tcf[pl.ds(j * 16, 16)] = thist[pl.ds(j * 16, 16)].astype(
                jnp.float32
            )
            return 0

        lax.fori_loop(0, nbins // 16, cvt, 0)
        pltpu.sync_copy(tcf, sh_all.at[pl.ds(768 * sid, _NB)])
        pltpu.sync_copy(tshist, sh_all.at[pl.ds(768 * sid + _NB, _NB)])
        plsc.subcore_barrier()
        pltpu.sync_copy(sh_all.at[pl.ds(768 * q0, 3072)], comb)
        plsc.subcore_barrier()

        def body(j, _):
            acc_c = zero_f
            acc_s = zero_f
            for r in range(4):
                acc_c = acc_c + comb[pl.ds(768 * r + j * 16, 16)]
                acc_s = acc_s + comb[pl.ds(768 * r + _NB + j * 16, 16)]
            thist[pl.ds(j * 16, 16)] = acc_c.astype(jnp.int32)
            tshist[pl.ds(j * 16, 16)] = acc_s
            return 0

        lax.fori_loop(0, nbins // 16, body, 0)

    def cnt_ge(e, nbins):
        def body(j, acc):
            lbl = j * 16 + iot
            return acc + jnp.where(lbl >= e, thist[pl.ds(j * 16, 16)], zero_i)

        return jnp.sum(lax.fori_loop(0, nbins // 16, body, zero_i))

    def sum_ge_vec(e, nbins):
        def body(j, acc):
            lbl = j * 16 + iot
            return acc + jnp.where(lbl >= e, tshist[pl.ds(j * 16, 16)], zero_f)

        return lax.fori_loop(0, nbins // 16, body, zero_f)

    def search(k_rem, nbits):
        nbins = 1 << nbits

        def body(_, c):
            lo, hi = c
            mid = lo + (hi - lo) // 2
            ok = cnt_ge(mid, nbins) >= k_rem
            return jnp.where(ok, mid, lo), jnp.where(ok, hi, mid)

        lo, _ = lax.fori_loop(
            0, nbits, body, (jnp.int32(0), jnp.int32(nbins))
        )
        return lo, cnt_ge(lo + 1, nbins), sum_ge_vec(lo + 1, nbins)

    lane_reduce(1 << _LBITS[0])
    statv[pl.ds(0, 16)] = ap.astype(jnp.float32)
    pltpu.sync_copy(statv, sh_all.at[pl.ds(768 * sid + 2 * _NB, _NB)])
    publish_combine(1 << _LBITS[0])

    pv = zero_f
    for r in range(4):
        pv = pv + comb[pl.ds(768 * r + 2 * _NB, 16)]
    p_i = jnp.sum(pv.astype(jnp.int32))
    k_i = 3 * p_i

    pos_sum = jnp.sum(jnp.where(iot == 0, tshist[pl.ds(0, 16)], zero_f))
    tacc = zero_f
    for j in range(16):
        tacc = tacc + tshist[pl.ds(j * 16, 16)]
    total = jnp.sum(tacc)

    b1, ac1, asv1 = search(k_i, _LBITS[0])
    prefix = b1
    k_rem = k_i - ac1
    above_cnt = ac1
    asum_v = asv1

    lane_base = iot * 4096
    sh1 = jnp.full((16,), _SHIFTS[0], jnp.int32)
    sh2 = jnp.full((16,), _SHIFTS[1], jnp.int32)
    sh3 = jnp.full((16,), _SHIFTS[2], jnp.int32)
    bm8 = jnp.full((16,), 255, jnp.int32)
    bm7 = jnp.full((16,), 127, jnp.int32)

    plsc.parallel_loop(0, _NB * 16, step=64)(zero_body)
    b1_v = jnp.broadcast_to(b1, (16,))

    def l2_body(j, offv):
        for u in range(_U):
            v = plsc.load_gather(nb, [lane_base + j + u])
            m = lax.shift_right_logical(v, sh1) == b1_v
            bn = jnp.bitwise_and(lax.shift_right_logical(v, sh2), bm8)
            idx = iot * _NB + bn
            plsc.addupdate_scatter(hist, [idx], ones_i, mask=m)
            plsc.addupdate_scatter(
                shist, [idx], lax.bitcast_convert_type(v, jnp.float32), mask=m
            )
            plsc.store_scatter(nb, [lane_base + offv], v, mask=m)
            offv = offv + jnp.where(m, ones_i, zero_i)
        return offv

    cntv = plsc.parallel_loop(0, 4096, step=_U, carry=zero_i)(l2_body)
    lane_reduce(_NB)
    publish_combine(_NB)
    b2, ac2, asv2 = search(k_rem, _LBITS[1])
    prefix = b1 * 256 + b2
    k_rem = k_rem - ac2
    above_cnt = above_cnt + ac2
    asum_v = asum_v + asv2

    maxc = jnp.max(cntv)
    plsc.parallel_loop(0, _NB * 16, step=64)(zero_body)
    p2_v = jnp.broadcast_to(prefix, (16,))

    def l3_body(j, offv):
        jv = jnp.broadcast_to(j, (16,))
        v = plsc.load_gather(nb, [lane_base + jv])
        m = (jv < cntv) & (lax.shift_right_logical(v, sh2) == p2_v)
        bn = jnp.bitwise_and(lax.shift_right_logical(v, sh3), bm8)
        idx = iot * _NB + bn
        plsc.addupdate_scatter(hist, [idx], ones_i, mask=m)
        plsc.addupdate_scatter(
            shist, [idx], lax.bitcast_convert_type(v, jnp.float32), mask=m
        )
        plsc.store_scatter(nb, [lane_base + offv], v, mask=m)
        return offv + jnp.where(m, ones_i, zero_i)

    cntv2 = plsc.parallel_loop(0, maxc, step=1, carry=zero_i)(l3_body)
    lane_reduce(_NB)
    publish_combine(_NB)
    b3, ac3, asv3 = search(k_rem, _LBITS[2])
    prefix = prefix * 256 + b3
    k_rem = k_rem - ac3
    above_cnt = above_cnt + ac3
    asum_v = asum_v + asv3

    maxc2 = jnp.max(cntv2)
    plsc.parallel_loop(0, _NB * 16, step=64)(zero_body)
    p3_v = jnp.broadcast_to(prefix, (16,))

    def l4_body(j):
        jv = jnp.broadcast_to(j, (16,))
        v = plsc.load_gather(nb, [lane_base + jv])
        m = (jv < cntv2) & (lax.shift_right_logical(v, sh3) == p3_v)
        bn = jnp.bitwise_and(v, bm7)
        idx = iot * _NB + bn
        plsc.addupdate_scatter(hist, [idx], ones_i, mask=m)
        plsc.addupdate_scatter(
            shist, [idx], lax.bitcast_convert_type(v, jnp.float32), mask=m
        )

    plsc.parallel_loop(0, maxc2, step=1)(l4_body)
    lane_reduce(128)
    publish_combine(128)
    b4, ac4, asv4 = search(k_rem, _LBITS[3])
    prefix = prefix * 128 + b4
    k_rem = k_rem - ac4
    above_cnt = above_cnt + ac4
    asum_v = asum_v + asv4

    t_vec = lax.bitcast_convert_type(jnp.broadcast_to(prefix, (16,)), jnp.float32)
    kf_v = jnp.broadcast_to(k_i, (16,)).astype(jnp.float32)
    pf_v = jnp.broadcast_to(p_i, (16,)).astype(jnp.float32)
    cgt_v = jnp.broadcast_to(above_cnt, (16,)).astype(jnp.float32)
    sum_gt_v = jnp.broadcast_to(jnp.sum(asum_v), (16,))
    pos_sum_v = jnp.broadcast_to(pos_sum, (16,))
    total_v = jnp.broadcast_to(total, (16,))

    topk_v = sum_gt_v + (kf_v - cgt_v) * t_vec
    fallback_v = total_v * (1.0 / _N)
    mined_v = pos_sum_v / jnp.maximum(pf_v, 1.0) + topk_v / jnp.maximum(
        kf_v, 1.0
    )
    cond = (k_i + p_i >= _N) | (k_i <= 10)
    outv[...] = jnp.where(cond, fallback_v, mined_v)

    @pl.when(quarter == 0)
    def _():
        pltpu.sync_copy(outv, out_hbm.at[sample])


_sc_kernel = functools.partial(
    pl.kernel,
    out_type=jax.ShapeDtypeStruct((_B, 16), jnp.float32),
    mesh=plsc.VectorSubcoreMesh(core_axis_name="c", subcore_axis_name="s"),
    compiler_params=pltpu.CompilerParams(needs_layout_passes=False),
    scratch_types=[
        pltpu.VMEM((_M,), jnp.int32),
        pltpu.VMEM((_CHUNK,), jnp.float32),
        pltpu.VMEM((_CHUNK,), jnp.float32),
        pltpu.VMEM((_NB * 16,), jnp.int32),
        pltpu.VMEM((_NB * 16,), jnp.float32),
        pltpu.VMEM((3072,), jnp.float32),
        pltpu.VMEM((_NB,), jnp.int32),
        pltpu.VMEM((_NB,), jnp.float32),
        pltpu.VMEM((_NB,), jnp.float32),
        pltpu.VMEM((_NB,), jnp.float32),
        pltpu.VMEM((16,), jnp.float32),
        pltpu.VMEM_SHARED((12288,), jnp.float32),
    ],
)(_sc_body)


def kernel(x, y):
    out = _sc_kernel(x.reshape(-1), y.reshape(-1))
    return jnp.mean(out[:, 0])


# speedup vs baseline: 1.3801x; 1.3801x over previous
import functools

import jax
import jax.numpy as jnp
from jax import lax
from jax.experimental import pallas as pl
from jax.experimental.pallas import tpu as pltpu
from jax.experimental.pallas import tpu_sc as plsc

_POS_TH = 2.0
_B = 8
_N = 512 * 512
_M = _N // 4
_CHUNK = 8192
_NCH = _M // _CHUNK
_CV = _CHUNK // 16
_NV = _M // 16
_NB = 256
_SHIFTS = (23, 15, 7, 0)
_LBITS = (8, 8, 8, 7)


def _sc_body(x_hbm, y_hbm, out_hbm, nb, xb, yb, hist, shist, comb,
             thist, tshist, tcf, statv, outv, sh_all):
    cid = lax.axis_index("c")
    sid = lax.axis_index("s")
    sample = cid * 4 + sid // 4
    quarter = sid % 4
    q0 = (sid // 4) * 4
    base = pl.multiple_of(sample * _N + quarter * _M, 8)

    iot = lax.iota(jnp.int32, 16)
    zero_i = jnp.zeros((16,), jnp.int32)
    zero_f = jnp.zeros((16,), jnp.float32)
    ones_i = jnp.ones((16,), jnp.int32)

    def zero_body(j):
        for u in range(4):
            hist[pl.ds(j + u * 16, 16)] = zero_i
            shist[pl.ds(j + u * 16, 16)] = zero_f

    plsc.parallel_loop(0, _NB * 16, step=64)(zero_body)

    _U = 4
    sh0 = jnp.full((16,), _SHIFTS[0], jnp.int32)
    apc = (zero_i,) * _U
    for c in range(_NCH):
        off = pl.multiple_of(base + c * _CHUNK, 8)
        pltpu.sync_copy(x_hbm.at[pl.ds(off, _CHUNK)], xb)
        pltpu.sync_copy(y_hbm.at[pl.ds(off, _CHUNK)], yb)

        def p1_body(i, acc, c=c):
            out = []
            for u in range(_U):
                s = i + u * 16
                xv = xb[pl.ds(s, 16)]
                yv = yb[pl.ds(s, 16)]
                d = xv - yv
                lv = d * d
                posm = yv > _POS_TH
                nbv = jnp.where(
                    posm, zero_i, lax.bitcast_convert_type(lv, jnp.int32)
                )
                nb[pl.ds(c * _CHUNK + s, 16)] = nbv
                idx = iot * _NB + lax.shift_right_logical(nbv, sh0)
                plsc.addupdate_scatter(hist, [idx], ones_i)
                plsc.addupdate_scatter(shist, [idx], lv)
                out.append(acc[u] + jnp.where(posm, ones_i, zero_i))
            return tuple(out)

        apc = plsc.parallel_loop(0, _CHUNK, step=16 * _U, carry=apc)(p1_body)
    ap = apc[0] + apc[1] + apc[2] + apc[3]

    def lane_reduce(nbins):
        def body(j, _):
            acc_c = zero_i
            acc_s = zero_f
            for l in range(16):
                acc_c = acc_c + hist[pl.ds(l * _NB + j * 16, 16)]
                acc_s = acc_s + shist[pl.ds(l * _NB + j * 16, 16)]
            thist[pl.ds(j * 16, 16)] = acc_c
            tshist[pl.ds(j * 16, 16)] = acc_s
            return 0

        lax.fori_loop(0, nbins // 16, body, 0)

    def publish_combine(nbins):
        def cvt(j, _):
            tcf[pl.ds(j * 16, 16)] = thist[pl.ds(j * 16, 16)].astype(
                jnp.float32
            )
            return 0

        lax.fori_loop(0, nbins // 16, cvt, 0)
        pltpu.sync_copy(tcf, sh_all.at[pl.ds(768 * sid, _NB)])
        pltpu.sync_copy(tshist, sh_all.at[pl.ds(768 * sid + _NB, _NB)])
        plsc.subcore_barrier()
        pltpu.sync_copy(sh_all.at[pl.ds(768 * q0, 3072)], comb)
        plsc.subcore_barrier()

        def body(j, _):
            acc_c = zero_f
            acc_s = zero_f
            for r in range(4):
                acc_c = acc_c + comb[pl.ds(768 * r + j * 16, 16)]
                acc_s = acc_s + comb[pl.ds(768 * r + _NB + j * 16, 16)]
            thist[pl.ds(j * 16, 16)] = acc_c.astype(jnp.int32)
            tshist[pl.ds(j * 16, 16)] = acc_s
            return 0

        lax.fori_loop(0, nbins // 16, body, 0)

    def cnt_ge(e, nbins):
        def body(j, acc):
            lbl = j * 16 + iot
            return acc + jnp.where(lbl >= e, thist[pl.ds(j * 16, 16)], zero_i)

        return jnp.sum(lax.fori_loop(0, nbins // 16, body, zero_i))

    def sum_ge_vec(e, nbins):
        def body(j, acc):
            lbl = j * 16 + iot
            return acc + jnp.where(lbl >= e, tshist[pl.ds(j * 16, 16)], zero_f)

        return lax.fori_loop(0, nbins // 16, body, zero_f)

    def search(k_rem, nbits):
        nbins = 1 << nbits

        def body(_, c):
            lo, hi = c
            mid = lo + (hi - lo) // 2
            ok = cnt_ge(mid, nbins) >= k_rem
            return jnp.where(ok, mid, lo), jnp.where(ok, hi, mid)

        lo, _ = lax.fori_loop(
            0, nbits, body, (jnp.int32(0), jnp.int32(nbins))
        )
        return lo, cnt_ge(lo + 1, nbins), sum_ge_vec(lo + 1, nbins)

    lane_reduce(1 << _LBITS[0])
    statv[pl.ds(0, 16)] = ap.astype(jnp.float32)
    pltpu.sync_copy(statv, sh_all.at[pl.ds(768 * sid + 2 * _NB, _NB)])
    publish_combine(1 << _LBITS[0])

    pv = zero_f
    for r in range(4):
        pv = pv + comb[pl.ds(768 * r + 2 * _NB, 16)]
    p_i = jnp.sum(pv.astype(jnp.int32))
    k_i = 3 * p_i

    pos_sum = jnp.sum(jnp.where(iot == 0, tshist[pl.ds(0, 16)], zero_f))
    tacc = zero_f
    for j in range(16):
        tacc = tacc + tshist[pl.ds(j * 16, 16)]
    total = jnp.sum(tacc)

    b1, ac1, asv1 = search(k_i, _LBITS[0])
    prefix = b1
    k_rem = k_i - ac1
    above_cnt = ac1
    asum_v = asv1

    for lvl in range(1, 4):
        nbits = _LBITS[lvl]
        nbins = 1 << nbits
        plsc.parallel_loop(0, _NB * 16, step=64)(zero_body)
        pv_prefix = jnp.broadcast_to(prefix, (16,))
        shp = jnp.full((16,), _SHIFTS[lvl - 1], jnp.int32)
        shc = jnp.full((16,), _SHIFTS[lvl], jnp.int32)
        bmask = jnp.full((16,), nbins - 1, jnp.int32)

        def hist_body(i):
            for u in range(_U):
                v = nb[pl.ds(i + u * 16, 16)]
                m = lax.shift_right_logical(v, shp) == pv_prefix
                bn = jnp.bitwise_and(lax.shift_right_logical(v, shc), bmask)
                idx = iot * _NB + bn
                plsc.addupdate_scatter(hist, [idx], ones_i, mask=m)
                plsc.addupdate_scatter(
                    shist,
                    [idx],
                    lax.bitcast_convert_type(v, jnp.float32),
                    mask=m,
                )

        plsc.parallel_loop(0, _M, step=16 * _U)(hist_body)
        lane_reduce(nbins)
        publish_combine(nbins)
        b, ac, asv = search(k_rem, nbits)
        prefix = prefix * nbins + b
        k_rem = k_rem - ac
        above_cnt = above_cnt + ac
        asum_v = asum_v + asv

    t_vec = lax.bitcast_convert_type(jnp.broadcast_to(prefix, (16,)), jnp.float32)
    kf_v = jnp.broadcast_to(k_i, (16,)).astype(jnp.float32)
    pf_v = jnp.broadcast_to(p_i, (16,)).astype(jnp.float32)
    cgt_v = jnp.broadcast_to(above_cnt, (16,)).astype(jnp.float32)
    sum_gt_v = jnp.broadcast_to(jnp.sum(asum_v), (16,))
    pos_sum_v = jnp.broadcast_to(pos_sum, (16,))
    total_v = jnp.broadcast_to(total, (16,))

    topk_v = sum_gt_v + (kf_v - cgt_v) * t_vec
    fallback_v = total_v * (1.0 / _N)
    mined_v = pos_sum_v / jnp.maximum(pf_v, 1.0) + topk_v / jnp.maximum(
        kf_v, 1.0
    )
    cond = (k_i + p_i >= _N) | (k_i <= 10)
    outv[...] = jnp.where(cond, fallback_v, mined_v)

    @pl.when(quarter == 0)
    def _():
        pltpu.sync_copy(outv, out_hbm.at[sample])


_sc_kernel = functools.partial(
    pl.kernel,
    out_type=jax.ShapeDtypeStruct((_B, 16), jnp.float32),
    mesh=plsc.VectorSubcoreMesh(core_axis_name="c", subcore_axis_name="s"),
    compiler_params=pltpu.CompilerParams(needs_layout_passes=False),
    scratch_types=[
        pltpu.VMEM((_M,), jnp.int32),
        pltpu.VMEM((_CHUNK,), jnp.float32),
        pltpu.VMEM((_CHUNK,), jnp.float32),
        pltpu.VMEM((_NB * 16,), jnp.int32),
        pltpu.VMEM((_NB * 16,), jnp.float32),
        pltpu.VMEM((3072,), jnp.float32),
        pltpu.VMEM((_NB,), jnp.int32),
        pltpu.VMEM((_NB,), jnp.float32),
        pltpu.VMEM((_NB,), jnp.float32),
        pltpu.VMEM((_NB,), jnp.float32),
        pltpu.VMEM((16,), jnp.float32),
        pltpu.VMEM_SHARED((12288,), jnp.float32),
    ],
)(_sc_body)


def kernel(x, y):
    out = _sc_kernel(x.reshape(-1), y.reshape(-1))
    return jnp.mean(out[:, 0])


# speedup vs baseline: 1.5014x; 1.0879x over previous
import functools

import jax
import jax.numpy as jnp
from jax import lax
from jax.experimental import pallas as pl
from jax.experimental.pallas import tpu as pltpu
from jax.experimental.pallas import tpu_sc as plsc

_POS_TH = 2.0
_B = 8
_N = 512 * 512
_M = _N // 4
_CHUNK = 16384
_NCH = _M // _CHUNK
_CV = _CHUNK // 16
_NV = _M // 16
_NB = 256
_SHIFTS = (23, 15, 7, 0)
_LBITS = (8, 8, 8, 7)


def _sc_body(x_hbm, y_hbm, out_hbm, nb, xb, yb, hist, shist, comb,
             thist, tshist, tcf, statv, outv, semx, semy, sh_all):
    cid = lax.axis_index("c")
    sid = lax.axis_index("s")
    sample = cid * 4 + sid // 4
    quarter = sid % 4
    q0 = (sid // 4) * 4
    base = pl.multiple_of(sample * _N + quarter * _M, 8)

    iot = lax.iota(jnp.int32, 16)
    zero_i = jnp.zeros((16,), jnp.int32)
    zero_f = jnp.zeros((16,), jnp.float32)
    ones_i = jnp.ones((16,), jnp.int32)

    def zero_body(j):
        for u in range(4):
            hist[pl.ds(j + u * 16, 16)] = zero_i
            shist[pl.ds(j + u * 16, 16)] = zero_f

    plsc.parallel_loop(0, _NB * 16, step=64)(zero_body)

    _U = 4
    sh0 = jnp.full((16,), _SHIFTS[0], jnp.int32)
    apc = (zero_i,) * _U
    for c in range(_NCH):
        off = pl.multiple_of(base + c * _CHUNK, 8)
        cpx = pltpu.async_copy(x_hbm.at[pl.ds(off, _CHUNK)], xb, semx)
        cpy = pltpu.async_copy(y_hbm.at[pl.ds(off, _CHUNK)], yb, semy)
        cpx.wait()
        cpy.wait()

        def p1_body(i, acc, c=c):
            out = []
            for u in range(_U):
                s = i + u * 16
                xv = xb[pl.ds(s, 16)]
                yv = yb[pl.ds(s, 16)]
                d = xv - yv
                lv = d * d
                posm = yv > _POS_TH
                nbv = jnp.where(
                    posm, zero_i, lax.bitcast_convert_type(lv, jnp.int32)
                )
                nb[pl.ds(c * _CHUNK + s, 16)] = nbv
                idx = iot * _NB + lax.shift_right_logical(nbv, sh0)
                plsc.addupdate_scatter(hist, [idx], ones_i)
                plsc.addupdate_scatter(shist, [idx], lv)
                out.append(acc[u] + jnp.where(posm, ones_i, zero_i))
            return tuple(out)

        apc = plsc.parallel_loop(0, _CHUNK, step=16 * _U, carry=apc)(p1_body)
    ap = apc[0] + apc[1] + apc[2] + apc[3]

    def lane_reduce(nbins):
        def body(j, _):
            acc_c = zero_i
            acc_s = zero_f
            for l in range(16):
                acc_c = acc_c + hist[pl.ds(l * _NB + j * 16, 16)]
                acc_s = acc_s + shist[pl.ds(l * _NB + j * 16, 16)]
            thist[pl.ds(j * 16, 16)] = acc_c
            tshist[pl.ds(j * 16, 16)] = acc_s
            return 0

        lax.fori_loop(0, nbins // 16, body, 0)

    def publish_combine(nbins):
        def cvt(j, _):
            tcf[pl.ds(j * 16, 16)] = thist[pl.ds(j * 16, 16)].astype(
                jnp.float32
            )
            return 0

        lax.fori_loop(0, nbins // 16, cvt, 0)
        pltpu.sync_copy(tcf, sh_all.at[pl.ds(768 * sid, _NB)])
        pltpu.sync_copy(tshist, sh_all.at[pl.ds(768 * sid + _NB, _NB)])
        plsc.subcore_barrier()
        pltpu.sync_copy(sh_all.at[pl.ds(768 * q0, 3072)], comb)
        plsc.subcore_barrier()

        def body(j, _):
            acc_c = zero_f
            acc_s = zero_f
            for r in range(4):
                acc_c = acc_c + comb[pl.ds(768 * r + j * 16, 16)]
                acc_s = acc_s + comb[pl.ds(768 * r + _NB + j * 16, 16)]
            thist[pl.ds(j * 16, 16)] = acc_c.astype(jnp.int32)
            tshist[pl.ds(j * 16, 16)] = acc_s
            return 0

        lax.fori_loop(0, nbins // 16, body, 0)

    def cnt_ge(e, nbins):
        def body(j, acc):
            lbl = j * 16 + iot
            return acc + jnp.where(lbl >= e, thist[pl.ds(j * 16, 16)], zero_i)

        return jnp.sum(lax.fori_loop(0, nbins // 16, body, zero_i))

    def sum_ge_vec(e, nbins):
        def body(j, acc):
            lbl = j * 16 + iot
            return acc + jnp.where(lbl >= e, tshist[pl.ds(j * 16, 16)], zero_f)

        return lax.fori_loop(0, nbins // 16, body, zero_f)

    def search(k_rem, nbits):
        nbins = 1 << nbits

        def body(_, c):
            lo, hi = c
            mid = lo + (hi - lo) // 2
            ok = cnt_ge(mid, nbins) >= k_rem
            return jnp.where(ok, mid, lo), jnp.where(ok, hi, mid)

        lo, _ = lax.fori_loop(
            0, nbits, body, (jnp.int32(0), jnp.int32(nbins))
        )
        return lo, cnt_ge(lo + 1, nbins), sum_ge_vec(lo + 1, nbins)

    lane_reduce(1 << _LBITS[0])
    statv[pl.ds(0, 16)] = ap.astype(jnp.float32)
    pltpu.sync_copy(statv, sh_all.at[pl.ds(768 * sid + 2 * _NB, _NB)])
    publish_combine(1 << _LBITS[0])

    pv = zero_f
    for r in range(4):
        pv = pv + comb[pl.ds(768 * r + 2 * _NB, 16)]
    p_i = jnp.sum(pv.astype(jnp.int32))
    k_i = 3 * p_i

    pos_sum = jnp.sum(jnp.where(iot == 0, tshist[pl.ds(0, 16)], zero_f))
    tacc = zero_f
    for j in range(16):
        tacc = tacc + tshist[pl.ds(j * 16, 16)]
    total = jnp.sum(tacc)

    b1, ac1, asv1 = search(k_i, _LBITS[0])
    prefix = b1
    k_rem = k_i - ac1
    above_cnt = ac1
    asum_v = asv1

    for lvl in range(1, 4):
        nbits = _LBITS[lvl]
        nbins = 1 << nbits
        plsc.parallel_loop(0, _NB * 16, step=64)(zero_body)
        pv_prefix = jnp.broadcast_to(prefix, (16,))
        shp = jnp.full((16,), _SHIFTS[lvl - 1], jnp.int32)
        shc = jnp.full((16,), _SHIFTS[lvl], jnp.int32)
        bmask = jnp.full((16,), nbins - 1, jnp.int32)

        def hist_body(i):
            for u in range(_U):
                v = nb[pl.ds(i + u * 16, 16)]
                m = lax.shift_right_logical(v, shp) == pv_prefix
                bn = jnp.bitwise_and(lax.shift_right_logical(v, shc), bmask)
                idx = iot * _NB + bn
                plsc.addupdate_scatter(hist, [idx], ones_i, mask=m)
                plsc.addupdate_scatter(
                    shist,
                    [idx],
                    lax.bitcast_convert_type(v, jnp.float32),
                    mask=m,
                )

        plsc.parallel_loop(0, _M, step=16 * _U)(hist_body)
        lane_reduce(nbins)
        publish_combine(nbins)
        b, ac, asv = search(k_rem, nbits)
        prefix = prefix * nbins + b
        k_rem = k_rem - ac
        above_cnt = above_cnt + ac
        asum_v = asum_v + asv

    t_vec = lax.bitcast_convert_type(jnp.broadcast_to(prefix, (16,)), jnp.float32)
    kf_v = jnp.broadcast_to(k_i, (16,)).astype(jnp.float32)
    pf_v = jnp.broadcast_to(p_i, (16,)).astype(jnp.float32)
    cgt_v = jnp.broadcast_to(above_cnt, (16,)).astype(jnp.float32)
    sum_gt_v = jnp.broadcast_to(jnp.sum(asum_v), (16,))
    pos_sum_v = jnp.broadcast_to(pos_sum, (16,))
    total_v = jnp.broadcast_to(total, (16,))

    topk_v = sum_gt_v + (kf_v - cgt_v) * t_vec
    fallback_v = total_v * (1.0 / _N)
    mined_v = pos_sum_v / jnp.maximum(pf_v, 1.0) + topk_v / jnp.maximum(
        kf_v, 1.0
    )
    cond = (k_i + p_i >= _N) | (k_i <= 10)
    outv[...] = jnp.where(cond, fallback_v, mined_v)

    @pl.when(quarter == 0)
    def _():
        pltpu.sync_copy(outv, out_hbm.at[sample])


_sc_kernel = functools.partial(
    pl.kernel,
    out_type=jax.ShapeDtypeStruct((_B, 16), jnp.float32),
    mesh=plsc.VectorSubcoreMesh(core_axis_name="c", subcore_axis_name="s"),
    compiler_params=pltpu.CompilerParams(needs_layout_passes=False),
    scratch_types=[
        pltpu.VMEM((_M,), jnp.int32),
        pltpu.VMEM((_CHUNK,), jnp.float32),
        pltpu.VMEM((_CHUNK,), jnp.float32),
        pltpu.VMEM((_NB * 16,), jnp.int32),
        pltpu.VMEM((_NB * 16,), jnp.float32),
        pltpu.VMEM((3072,), jnp.float32),
        pltpu.VMEM((_NB,), jnp.int32),
        pltpu.VMEM((_NB,), jnp.float32),
        pltpu.VMEM((_NB,), jnp.float32),
        pltpu.VMEM((_NB,), jnp.float32),
        pltpu.VMEM((16,), jnp.float32),
        pltpu.SemaphoreType.DMA,
        pltpu.SemaphoreType.DMA,
        pltpu.VMEM_SHARED((12288,), jnp.float32),
    ],
)(_sc_body)


def kernel(x, y):
    out = _sc_kernel(x.reshape(-1), y.reshape(-1))
    return jnp.mean(out[:, 0])


# speedup vs baseline: 1.5730x; 1.0477x over previous
import functools

import jax
import jax.numpy as jnp
from jax import lax
from jax.experimental import pallas as pl
from jax.experimental.pallas import tpu as pltpu
from jax.experimental.pallas import tpu_sc as plsc

_POS_TH = 2.0
_B = 8
_N = 512 * 512
_M = _N // 4
_CHUNK = 8192
_NCH = _M // _CHUNK
_CV = _CHUNK // 16
_NV = _M // 16
_NB = 256
_SHIFTS = (23, 15, 7, 0)
_LBITS = (8, 8, 8, 7)


def _sc_body(x_hbm, y_hbm, out_hbm, nb, xb, yb, hist, shist, comb,
             thist, tshist, tcf, statv, outv, semx, semy, semx2, semy2,
             sh_all):
    cid = lax.axis_index("c")
    sid = lax.axis_index("s")
    sample = cid * 4 + sid // 4
    quarter = sid % 4
    q0 = (sid // 4) * 4
    base = pl.multiple_of(sample * _N + quarter * _M, 8)

    iot = lax.iota(jnp.int32, 16)
    zero_i = jnp.zeros((16,), jnp.int32)
    zero_f = jnp.zeros((16,), jnp.float32)
    ones_i = jnp.ones((16,), jnp.int32)

    def zero_body(j):
        for u in range(4):
            hist[pl.ds(j + u * 16, 16)] = zero_i
            shist[pl.ds(j + u * 16, 16)] = zero_f

    plsc.parallel_loop(0, _NB * 16, step=64)(zero_body)

    _U = 4
    sh0 = jnp.full((16,), _SHIFTS[0], jnp.int32)
    apc = (zero_i,) * _U

    def start_chunk(c):
        off = pl.multiple_of(base + c * _CHUNK, 8)
        b = (c % 2) * _CHUNK
        sx = semx if c % 2 == 0 else semx2
        sy = semy if c % 2 == 0 else semy2
        return (
            pltpu.async_copy(
                x_hbm.at[pl.ds(off, _CHUNK)], xb.at[pl.ds(b, _CHUNK)], sx
            ),
            pltpu.async_copy(
                y_hbm.at[pl.ds(off, _CHUNK)], yb.at[pl.ds(b, _CHUNK)], sy
            ),
        )

    cps = start_chunk(0)
    for c in range(_NCH):
        cps[0].wait()
        cps[1].wait()
        if c + 1 < _NCH:
            cps = start_chunk(c + 1)
        bb = (c % 2) * _CHUNK

        def p1_body(i, acc, c=c, bb=bb):
            out = []
            for u in range(_U):
                s = i + u * 16
                xv = xb[pl.ds(bb + s, 16)]
                yv = yb[pl.ds(bb + s, 16)]
                d = xv - yv
                lv = d * d
                posm = yv > _POS_TH
                nbv = jnp.where(
                    posm, zero_i, lax.bitcast_convert_type(lv, jnp.int32)
                )
                nb[pl.ds(c * _CHUNK + s, 16)] = nbv
                idx = iot * _NB + lax.shift_right_logical(nbv, sh0)
                plsc.addupdate_scatter(hist, [idx], ones_i)
                plsc.addupdate_scatter(shist, [idx], lv)
                out.append(acc[u] + jnp.where(posm, ones_i, zero_i))
            return tuple(out)

        apc = plsc.parallel_loop(0, _CHUNK, step=16 * _U, carry=apc)(p1_body)
    ap = apc[0] + apc[1] + apc[2] + apc[3]

    def lane_reduce(nbins):
        def body(j, _):
            acc_c = zero_i
            acc_s = zero_f
            for l in range(16):
                acc_c = acc_c + hist[pl.ds(l * _NB + j * 16, 16)]
                acc_s = acc_s + shist[pl.ds(l * _NB + j * 16, 16)]
            thist[pl.ds(j * 16, 16)] = acc_c
            tshist[pl.ds(j * 16, 16)] = acc_s
            return 0

        lax.fori_loop(0, nbins // 16, body, 0)

    def publish_combine(nbins):
        def cvt(j, _):
            tcf[pl.ds(j * 16, 16)] = thist[pl.ds(j * 16, 16)].astype(
                jnp.float32
            )
            return 0

        lax.fori_loop(0, nbins // 16, cvt, 0)
        pltpu.sync_copy(tcf, sh_all.at[pl.ds(768 * sid, _NB)])
        pltpu.sync_copy(tshist, sh_all.at[pl.ds(768 * sid + _NB, _NB)])
        plsc.subcore_barrier()
        pltpu.sync_copy(sh_all.at[pl.ds(768 * q0, 3072)], comb)
        plsc.subcore_barrier()

        def body(j, _):
            acc_c = zero_f
            acc_s = zero_f
            for r in range(4):
                acc_c = acc_c + comb[pl.ds(768 * r + j * 16, 16)]
                acc_s = acc_s + comb[pl.ds(768 * r + _NB + j * 16, 16)]
            thist[pl.ds(j * 16, 16)] = acc_c.astype(jnp.int32)
            tshist[pl.ds(j * 16, 16)] = acc_s
            return 0

        lax.fori_loop(0, nbins // 16, body, 0)

    def cnt_ge(e, nbins):
        def body(j, acc):
            lbl = j * 16 + iot
            return acc + jnp.where(lbl >= e, thist[pl.ds(j * 16, 16)], zero_i)

        return jnp.sum(lax.fori_loop(0, nbins // 16, body, zero_i))

    def sum_ge_vec(e, nbins):
        def body(j, acc):
            lbl = j * 16 + iot
            return acc + jnp.where(lbl >= e, tshist[pl.ds(j * 16, 16)], zero_f)

        return lax.fori_loop(0, nbins // 16, body, zero_f)

    def search(k_rem, nbits):
        nbins = 1 << nbits

        def body(_, c):
            lo, hi = c
            mid = lo + (hi - lo) // 2
            ok = cnt_ge(mid, nbins) >= k_rem
            return jnp.where(ok, mid, lo), jnp.where(ok, hi, mid)

        lo, _ = lax.fori_loop(
            0, nbits, body, (jnp.int32(0), jnp.int32(nbins))
        )
        return lo, cnt_ge(lo + 1, nbins), sum_ge_vec(lo + 1, nbins)

    lane_reduce(1 << _LBITS[0])
    statv[pl.ds(0, 16)] = ap.astype(jnp.float32)
    pltpu.sync_copy(statv, sh_all.at[pl.ds(768 * sid + 2 * _NB, _NB)])
    publish_combine(1 << _LBITS[0])

    pv = zero_f
    for r in range(4):
        pv = pv + comb[pl.ds(768 * r + 2 * _NB, 16)]
    p_i = jnp.sum(pv.astype(jnp.int32))
    k_i = 3 * p_i

    pos_sum = jnp.sum(jnp.where(iot == 0, tshist[pl.ds(0, 16)], zero_f))
    tacc = zero_f
    for j in range(16):
        tacc = tacc + tshist[pl.ds(j * 16, 16)]
    total = jnp.sum(tacc)

    b1, ac1, asv1 = search(k_i, _LBITS[0])
    prefix = b1
    k_rem = k_i - ac1
    above_cnt = ac1
    asum_v = asv1

    for lvl in range(1, 4):
        nbits = _LBITS[lvl]
        nbins = 1 << nbits
        plsc.parallel_loop(0, _NB * 16, step=64)(zero_body)
        pv_prefix = jnp.broadcast_to(prefix, (16,))
        shp = jnp.full((16,), _SHIFTS[lvl - 1], jnp.int32)
        shc = jnp.full((16,), _SHIFTS[lvl], jnp.int32)
        bmask = jnp.full((16,), nbins - 1, jnp.int32)

        def hist_body(i):
            for u in range(_U):
                v = nb[pl.ds(i + u * 16, 16)]
                m = lax.shift_right_logical(v, shp) == pv_prefix
                bn = jnp.bitwise_and(lax.shift_right_logical(v, shc), bmask)
                idx = iot * _NB + bn
                plsc.addupdate_scatter(hist, [idx], ones_i, mask=m)
                plsc.addupdate_scatter(
                    shist,
                    [idx],
                    lax.bitcast_convert_type(v, jnp.float32),
                    mask=m,
                )

        plsc.parallel_loop(0, _M, step=16 * _U)(hist_body)
        lane_reduce(nbins)
        publish_combine(nbins)
        b, ac, asv = search(k_rem, nbits)
        prefix = prefix * nbins + b
        k_rem = k_rem - ac
        above_cnt = above_cnt + ac
        asum_v = asum_v + asv

    t_vec = lax.bitcast_convert_type(jnp.broadcast_to(prefix, (16,)), jnp.float32)
    kf_v = jnp.broadcast_to(k_i, (16,)).astype(jnp.float32)
    pf_v = jnp.broadcast_to(p_i, (16,)).astype(jnp.float32)
    cgt_v = jnp.broadcast_to(above_cnt, (16,)).astype(jnp.float32)
    sum_gt_v = jnp.broadcast_to(jnp.sum(asum_v), (16,))
    pos_sum_v = jnp.broadcast_to(pos_sum, (16,))
    total_v = jnp.broadcast_to(total, (16,))

    topk_v = sum_gt_v + (kf_v - cgt_v) * t_vec
    fallback_v = total_v * (1.0 / _N)
    mined_v = pos_sum_v / jnp.maximum(pf_v, 1.0) + topk_v / jnp.maximum(
        kf_v, 1.0
    )
    cond = (k_i + p_i >= _N) | (k_i <= 10)
    outv[...] = jnp.where(cond, fallback_v, mined_v)

    @pl.when(quarter == 0)
    def _():
        pltpu.sync_copy(outv, out_hbm.at[sample])


_sc_kernel = functools.partial(
    pl.kernel,
    out_type=jax.ShapeDtypeStruct((_B, 16), jnp.float32),
    mesh=plsc.VectorSubcoreMesh(core_axis_name="c", subcore_axis_name="s"),
    compiler_params=pltpu.CompilerParams(needs_layout_passes=False),
    scratch_types=[
        pltpu.VMEM((_M,), jnp.int32),
        pltpu.VMEM((2 * _CHUNK,), jnp.float32),
        pltpu.VMEM((2 * _CHUNK,), jnp.float32),
        pltpu.VMEM((_NB * 16,), jnp.int32),
        pltpu.VMEM((_NB * 16,), jnp.float32),
        pltpu.VMEM((3072,), jnp.float32),
        pltpu.VMEM((_NB,), jnp.int32),
        pltpu.VMEM((_NB,), jnp.float32),
        pltpu.VMEM((_NB,), jnp.float32),
        pltpu.VMEM((_NB,), jnp.float32),
        pltpu.VMEM((16,), jnp.float32),
        pltpu.SemaphoreType.DMA,
        pltpu.SemaphoreType.DMA,
        pltpu.SemaphoreType.DMA,
        pltpu.SemaphoreType.DMA,
        pltpu.VMEM_SHARED((12288,), jnp.float32),
    ],
)(_sc_body)


def kernel(x, y):
    out = _sc_kernel(x.reshape(-1), y.reshape(-1))
    return jnp.mean(out[:, 0])
